# baseline (speedup 1.0000x reference)
import jax
import jax.numpy as jnp
from jax.experimental import pallas as pl
from jax.experimental.pallas import tpu as pltpu
from jax.experimental.pallas import tpu_sc as plsc

B = 128
N = 48
DH = 128
DK = 2

_EDGE_R = 8192
_EDGE_CLS = 16
_NODE_CLS = 64
_NODE_OFFS = (0, 22, 28, 34, 37, 40)
_NODE_PADS = (0, 0, 2, 0, 0, 0)


def _edge_body(idx_ref, w_ref, out_ref):
    idx = idx_ref[0]
    ciota = jax.lax.broadcasted_iota(jnp.int32, (2 * _EDGE_CLS, _EDGE_R), 0)
    oh_t = ((ciota == idx).astype(jnp.float32)
            + (ciota == idx + _EDGE_CLS).astype(jnp.float32)
            * jnp.float32(2.0 ** -8)).astype(jnp.bfloat16)
    out_ref[...] = jax.lax.dot_general(
        oh_t, w_ref[...], (((0,), (0,)), ((), ())),
        preferred_element_type=jnp.float32)


def _node_body(i0, i1, i2, i3, i4, i5, w_ref, out_ref):
    m = out_ref.shape[0]
    ciota = jax.lax.broadcasted_iota(jnp.int32, (_NODE_CLS, m), 0)
    mh_t = jnp.zeros((_NODE_CLS, m), jnp.float32)
    for off, ref in zip(_NODE_OFFS, (i0, i1, i2, i3, i4, i5)):
        mh_t = mh_t + (ciota == (ref[0] + off)).astype(jnp.float32)
    out_ref[...] = jax.lax.dot_general(
        mh_t, w_ref[...], (((0,), (0,)), ((), ())),
        preferred_element_type=jnp.float32,
        precision=jax.lax.Precision.HIGHEST)


def kernel(node_type, hydrogens, charge, is_in_ring, is_aromatic, chirality,
           edge_type, W_node_type, W_hydrogens, W_charge, W_is_in_ring,
           W_is_aromatic, W_chirality, W_edge_type):
    w_edge = W_edge_type.at[0].set(0.0)
    w2 = w_edge.reshape(6, DH, DK).transpose(0, 2, 1).reshape(6 * DK, DH)
    w2 = jnp.pad(w2, ((0, _EDGE_CLS - 6 * DK), (0, 0)))
    w2_bits = jax.lax.bitcast_convert_type(w2, jnp.uint32)
    w2_hi_f32 = jax.lax.bitcast_convert_type(
        w2_bits & jnp.uint32(0xFFFF0000), jnp.float32)
    w2_hi = w2_hi_f32.astype(jnp.bfloat16)
    w2_lo = ((w2 - w2_hi_f32) * 256.0).astype(jnp.bfloat16)
    w2_hilo = jnp.concatenate([w2_hi, w2_lo], axis=0)

    node_ws = (W_node_type, W_hydrogens, W_charge, W_is_in_ring,
               W_is_aromatic, W_chirality)
    rows = [w.at[p].set(0.0) for w, p in zip(node_ws, _NODE_PADS)]
    w_node = jnp.concatenate(rows, axis=0)
    w_node = jnp.pad(w_node, ((0, _NODE_CLS - w_node.shape[0]), (0, 0)))

    ne = B * N * N
    nr = ne * DK
    idx2 = (edge_type.reshape(ne, 1).astype(jnp.int32) * DK
            + jnp.arange(DK, dtype=jnp.int32)[None, :]).reshape(-1)
    win = 128
    idx2 = idx2.reshape(1, nr)

    @pl.kernel(
        out_type=jax.ShapeDtypeStruct((nr, DH), jnp.float32),
        mesh=plsc.VectorSubcoreMesh(core_axis_name="core",
                                    subcore_axis_name="subcore"))
    def _sc_gather(w_hbm, i_hbm, o_hbm):
        def body(i_vmem, o_vmem):
            pltpu.sync_copy(w_hbm.at[i_vmem.at[0]], o_vmem)

        pltpu.emit_pipeline(
            body,
            grid=(nr // win,),
            in_specs=[pl.BlockSpec((1, win), lambda i: (0, i))],
            out_specs=[pl.BlockSpec((win, DH), lambda i: (i, 0))],
            core_axis_name=("core", "subcore"),
            dimension_semantics=(pltpu.PARALLEL,),
        )(i_hbm, o_hbm)

    edge_out = _sc_gather(w2, idx2)
    edge_out = jnp.swapaxes(edge_out.reshape(B, N, N, DK, DH), 3, 4)

    m = B * N
    nidx = [a.reshape(1, 1, m).astype(jnp.int32)
            for a in (node_type, hydrogens, charge, is_in_ring,
                      is_aromatic, chirality)]
    node_out = pl.pallas_call(
        _node_body,
        in_specs=[pl.BlockSpec((1, 1, m), lambda: (0, 0, 0))] * 6
        + [pl.BlockSpec((_NODE_CLS, DH), lambda: (0, 0))],
        out_specs=pl.BlockSpec((m, DH), lambda: (0, 0)),
        out_shape=jax.ShapeDtypeStruct((m, DH), jnp.float32),
    )(*nidx, w_node)
    node_out = node_out.reshape(B, N, DH)

    return (node_out, edge_out)


# speedup vs baseline: 1.0005x; 1.0005x over previous
import jax
import jax.numpy as jnp
from jax.experimental import pallas as pl
from jax.experimental.pallas import tpu as pltpu
from jax.experimental.pallas import tpu_sc as plsc

B = 128
N = 48
DH = 128
DK = 2

_EDGE_R = 8192
_EDGE_CLS = 16
_NODE_CLS = 64
_NODE_OFFS = (0, 22, 28, 34, 37, 40)
_NODE_PADS = (0, 0, 2, 0, 0, 0)


def _edge_body(idx_ref, w_ref, out_ref):
    idx = idx_ref[0]
    ciota = jax.lax.broadcasted_iota(jnp.int32, (2 * _EDGE_CLS, _EDGE_R), 0)
    oh_t = ((ciota == idx).astype(jnp.float32)
            + (ciota == idx + _EDGE_CLS).astype(jnp.float32)
            * jnp.float32(2.0 ** -8)).astype(jnp.bfloat16)
    out_ref[...] = jax.lax.dot_general(
        oh_t, w_ref[...], (((0,), (0,)), ((), ())),
        preferred_element_type=jnp.float32)


def _node_body(i0, i1, i2, i3, i4, i5, w_ref, out_ref):
    m = out_ref.shape[0]
    ciota = jax.lax.broadcasted_iota(jnp.int32, (_NODE_CLS, m), 0)
    mh_t = jnp.zeros((_NODE_CLS, m), jnp.float32)
    for off, ref in zip(_NODE_OFFS, (i0, i1, i2, i3, i4, i5)):
        mh_t = mh_t + (ciota == (ref[0] + off)).astype(jnp.float32)
    out_ref[...] = jax.lax.dot_general(
        mh_t, w_ref[...], (((0,), (0,)), ((), ())),
        preferred_element_type=jnp.float32,
        precision=jax.lax.Precision.HIGHEST)


def kernel(node_type, hydrogens, charge, is_in_ring, is_aromatic, chirality,
           edge_type, W_node_type, W_hydrogens, W_charge, W_is_in_ring,
           W_is_aromatic, W_chirality, W_edge_type):
    w_edge = W_edge_type.at[0].set(0.0)
    w2 = w_edge.reshape(6, DH, DK).transpose(0, 2, 1).reshape(6 * DK, DH)
    w2 = jnp.pad(w2, ((0, _EDGE_CLS - 6 * DK), (0, 0)))
    w2_bits = jax.lax.bitcast_convert_type(w2, jnp.uint32)
    w2_hi_f32 = jax.lax.bitcast_convert_type(
        w2_bits & jnp.uint32(0xFFFF0000), jnp.float32)
    w2_hi = w2_hi_f32.astype(jnp.bfloat16)
    w2_lo = ((w2 - w2_hi_f32) * 256.0).astype(jnp.bfloat16)
    w2_hilo = jnp.concatenate([w2_hi, w2_lo], axis=0)

    node_ws = (W_node_type, W_hydrogens, W_charge, W_is_in_ring,
               W_is_aromatic, W_chirality)
    rows = [w.at[p].set(0.0) for w, p in zip(node_ws, _NODE_PADS)]
    w_node = jnp.concatenate(rows, axis=0)
    w_node = jnp.pad(w_node, ((0, _NODE_CLS - w_node.shape[0]), (0, 0)))

    ne = B * N * N
    nr = ne * DK
    idx2 = (edge_type.reshape(ne, 1).astype(jnp.int32) * DK
            + jnp.arange(DK, dtype=jnp.int32)[None, :]).reshape(-1)
    win = 256
    idx2 = idx2.reshape(1, nr)

    @pl.kernel(
        out_type=jax.ShapeDtypeStruct((nr, DH), jnp.float32),
        mesh=plsc.VectorSubcoreMesh(core_axis_name="core",
                                    subcore_axis_name="subcore"))
    def _sc_gather(w_hbm, i_hbm, o_hbm):
        def body(i_vmem, o_vmem):
            pltpu.sync_copy(w_hbm.at[i_vmem.at[0]], o_vmem)

        pltpu.emit_pipeline(
            body,
            grid=(nr // win,),
            in_specs=[pl.BlockSpec((1, win), lambda i: (0, i))],
            out_specs=[pl.BlockSpec((win, DH), lambda i: (i, 0))],
            core_axis_name=("core", "subcore"),
            dimension_semantics=(pltpu.PARALLEL,),
        )(i_hbm, o_hbm)

    edge_out = _sc_gather(w2, idx2)
    edge_out = jnp.swapaxes(edge_out.reshape(B, N, N, DK, DH), 3, 4)

    m = B * N
    nidx = [a.reshape(1, 1, m).astype(jnp.int32)
            for a in (node_type, hydrogens, charge, is_in_ring,
                      is_aromatic, chirality)]
    node_out = pl.pallas_call(
        _node_body,
        in_specs=[pl.BlockSpec((1, 1, m), lambda: (0, 0, 0))] * 6
        + [pl.BlockSpec((_NODE_CLS, DH), lambda: (0, 0))],
        out_specs=pl.BlockSpec((m, DH), lambda: (0, 0)),
        out_shape=jax.ShapeDtypeStruct((m, DH), jnp.float32),
    )(*nidx, w_node)
    node_out = node_out.reshape(B, N, DH)

    return (node_out, edge_out)


# speedup vs baseline: 20.1058x; 20.0949x over previous
import jax
import jax.numpy as jnp
from jax.experimental import pallas as pl
from jax.experimental.pallas import tpu as pltpu

B = 128
N = 48
DH = 128
DK = 2

_EDGE_R = 12288
_EDGE_CLS = 16
_NODE_CLS = 64
_NODE_OFFS = (0, 22, 28, 34, 37, 40)
_NODE_PADS = (0, 0, 2, 0, 0, 0)


def _edge_body(idx_ref, w_ref, out_ref):
    idx = idx_ref[0]
    ciota = jax.lax.broadcasted_iota(jnp.int32, (2 * _EDGE_CLS, _EDGE_R), 0)
    oh_t = ((ciota == idx).astype(jnp.float32)
            + (ciota == idx + _EDGE_CLS).astype(jnp.float32)
            * jnp.float32(2.0 ** -8)).astype(jnp.bfloat16)
    out_ref[...] = jax.lax.dot_general(
        oh_t, w_ref[...], (((0,), (0,)), ((), ())),
        preferred_element_type=jnp.float32)


def _node_body(i0, i1, i2, i3, i4, i5, w_ref, out_ref):
    m = out_ref.shape[0]
    ciota = jax.lax.broadcasted_iota(jnp.int32, (_NODE_CLS, m), 0)
    mh_t = jnp.zeros((_NODE_CLS, m), jnp.float32)
    for off, ref in zip(_NODE_OFFS, (i0, i1, i2, i3, i4, i5)):
        mh_t = mh_t + (ciota == (ref[0] + off)).astype(jnp.float32)
    out_ref[...] = jax.lax.dot_general(
        mh_t, w_ref[...], (((0,), (0,)), ((), ())),
        preferred_element_type=jnp.float32,
        precision=jax.lax.Precision.HIGHEST)


def kernel(node_type, hydrogens, charge, is_in_ring, is_aromatic, chirality,
           edge_type, W_node_type, W_hydrogens, W_charge, W_is_in_ring,
           W_is_aromatic, W_chirality, W_edge_type):
    w_edge = W_edge_type.at[0].set(0.0)
    w2 = w_edge.reshape(6, DH, DK).transpose(0, 2, 1).reshape(6 * DK, DH)
    w2 = jnp.pad(w2, ((0, _EDGE_CLS - 6 * DK), (0, 0)))
    w2_bits = jax.lax.bitcast_convert_type(w2, jnp.uint32)
    w2_hi_f32 = jax.lax.bitcast_convert_type(
        w2_bits & jnp.uint32(0xFFFF0000), jnp.float32)
    w2_hi = w2_hi_f32.astype(jnp.bfloat16)
    w2_lo = ((w2 - w2_hi_f32) * 256.0).astype(jnp.bfloat16)
    w2_hilo = jnp.concatenate([w2_hi, w2_lo], axis=0)

    node_ws = (W_node_type, W_hydrogens, W_charge, W_is_in_ring,
               W_is_aromatic, W_chirality)
    rows = [w.at[p].set(0.0) for w, p in zip(node_ws, _NODE_PADS)]
    w_node = jnp.concatenate(rows, axis=0)
    w_node = jnp.pad(w_node, ((0, _NODE_CLS - w_node.shape[0]), (0, 0)))

    ne = B * N * N
    nr = ne * DK
    idx2 = (edge_type.reshape(ne, 1).astype(jnp.int32) * DK
            + jnp.arange(DK, dtype=jnp.int32)[None, :]).reshape(-1)
    grid = nr // _EDGE_R
    idx2 = idx2.reshape(grid, 1, _EDGE_R)
    edge_out = pl.pallas_call(
        _edge_body,
        grid=(grid,),
        in_specs=[
            pl.BlockSpec((1, 1, _EDGE_R), lambda i: (i, 0, 0)),
            pl.BlockSpec((2 * _EDGE_CLS, DH), lambda i: (0, 0)),
        ],
        out_specs=pl.BlockSpec((_EDGE_R, DH), lambda i: (i, 0)),
        out_shape=jax.ShapeDtypeStruct((nr, DH), jnp.float32),
        compiler_params=pltpu.CompilerParams(
            dimension_semantics=("parallel",)),
    )(idx2, w2_hilo)
    edge_out = jnp.swapaxes(edge_out.reshape(B, N, N, DK, DH), 3, 4)

    m = B * N
    nidx = [a.reshape(1, 1, m).astype(jnp.int32)
            for a in (node_type, hydrogens, charge, is_in_ring,
                      is_aromatic, chirality)]
    node_out = pl.pallas_call(
        _node_body,
        in_specs=[pl.BlockSpec((1, 1, m), lambda: (0, 0, 0))] * 6
        + [pl.BlockSpec((_NODE_CLS, DH), lambda: (0, 0))],
        out_specs=pl.BlockSpec((m, DH), lambda: (0, 0)),
        out_shape=jax.ShapeDtypeStruct((m, DH), jnp.float32),
    )(*nidx, w_node)
    node_out = node_out.reshape(B, N, DH)

    return (node_out, edge_out)
